# Initial kernel scaffold; baseline (speedup 1.0000x reference)
import jax
import jax.numpy as jnp
from jax.experimental import pallas as pl


def kernel(adj, h0, degree, edge_index, params):
    raise NotImplementedError("write your pallas kernel here")


# speedup vs baseline: 1.0324x; 1.0324x over previous
import functools

import jax
import jax.numpy as jnp
from jax import lax
from jax.experimental import pallas as pl
from jax.experimental.pallas import tpu as pltpu

N = 10000
F = 128
H1 = 128
H2 = 64
E = 160000

BM = 400
BR = 1000


def _gin_body(adj_ref, hfull_ref, hrow_ref, w1_ref, b1_ref, w2_ref, b2_ref,
              out_ref, *, final_relu):
    acc = jnp.dot(adj_ref[...], hfull_ref[...],
                  preferred_element_type=jnp.float32)
    agg = acc + hrow_ref[...]
    t = jnp.maximum(
        jnp.dot(agg, w1_ref[...], preferred_element_type=jnp.float32)
        + b1_ref[...], 0.0)
    o = (jnp.dot(t, w2_ref[...], preferred_element_type=jnp.float32)
         + b2_ref[...])
    if final_relu:
        o = jnp.maximum(o, 0.0)
    out_ref[...] = o


def _gin_layer(adj, h, w1, b1, w2, b2, final_relu):
    d_in = h.shape[1]
    d_hid = w1.shape[1]
    d_out = w2.shape[1]
    return pl.pallas_call(
        functools.partial(_gin_body, final_relu=final_relu),
        grid=(N // BM,),
        in_specs=[
            pl.BlockSpec((BM, N), lambda i: (i, 0)),
            pl.BlockSpec((N, d_in), lambda i: (0, 0)),
            pl.BlockSpec((BM, d_in), lambda i: (i, 0)),
            pl.BlockSpec((d_in, d_hid), lambda i: (0, 0)),
            pl.BlockSpec((1, d_hid), lambda i: (0, 0)),
            pl.BlockSpec((d_hid, d_out), lambda i: (0, 0)),
            pl.BlockSpec((1, d_out), lambda i: (0, 0)),
        ],
        out_specs=pl.BlockSpec((BM, d_out), lambda i: (i, 0)),
        out_shape=jax.ShapeDtypeStruct((N, d_out), jnp.float32),
    )(adj, h, h, w1, b1.reshape(1, -1), w2, b2.reshape(1, -1))


def _segment_sums(h0, h1, row, col):
    s0 = jax.ops.segment_sum(h0[col], row, num_segments=N)
    s1 = jax.ops.segment_sum(h1[col], row, num_segments=N)
    cnt = jax.ops.segment_sum(jnp.ones((E,), jnp.float32), row, num_segments=N)
    return s0, s1, cnt.reshape(N, 1)


def _mlp2_k(x, w1, b1, w2, b2):
    t = jnp.maximum(jnp.dot(x, w1, preferred_element_type=jnp.float32) + b1, 0.0)
    return jnp.dot(t, w2, preferred_element_type=jnp.float32) + b2


def _decoder_body(h0_ref, h1_ref, h2_ref, s0_ref, s1_ref, cnt_ref, n1_ref,
                  n2_ref, deg_ref,
                  rs0w1, rs0b1, rs0w2, rs0b2,
                  ds0w1, ds0b1, ds0w2, ds0b2,
                  rd0w1, rd0b1, rd0w2, rd0b2, rd0w3, rd0b3,
                  rs1w1, rs1b1, rs1w2, rs1b2,
                  ds1w1, ds1b1, ds1w2, ds1b2,
                  dm0w1, dm0b1, dm0w2, dm0b2,
                  rd1w1, rd1b1, rd1w2, rd1b2, rd1w3, rd1b3,
                  self_ref, kl_ref, deg_out_ref):
    i = pl.program_id(0)
    h0 = h0_ref[...]
    h1 = h1_ref[...]
    h2 = h2_ref[...]
    inv_c = 1.0 / (1.0 + cnt_ref[...])
    deg = deg_ref[...]

    mean1 = _mlp2_k(h2, rs0w1[...], rs0b1[...], rs0w2[...], rs0b2[...])
    ls1 = _mlp2_k(h2, ds0w1[...], ds0b1[...], ds0w2[...], ds0b2[...])
    z1 = mean1 + n1_ref[...] * jnp.exp(ls1)
    s_self = jnp.sum((h1 - z1) ** 2)
    mt1 = (h1 + s1_ref[...]) * inv_c
    s_kl = jnp.sum(-1.0 - 2.0 * ls1 + (mean1 - mt1) ** 2 + jnp.exp(2.0 * ls1))
    t = jnp.maximum(jnp.dot(h2, rd0w1[...], preferred_element_type=jnp.float32)
                    + rd0b1[...], 0.0)
    t = jnp.maximum(jnp.dot(t, rd0w2[...], preferred_element_type=jnp.float32)
                    + rd0b2[...], 0.0)
    rd = jnp.maximum(
        jnp.sum(t * rd0w3[...], axis=1, keepdims=True) + rd0b3[...], 0.0)
    s_deg = jnp.sum((rd - deg) ** 2)

    mean0 = _mlp2_k(h1, rs1w1[...], rs1b1[...], rs1w2[...], rs1b2[...])
    mprior = _mlp2_k(z1, dm0w1[...], dm0b1[...], dm0w2[...], dm0b2[...])
    mpost = mean0 + mprior
    ls0 = _mlp2_k(h1, ds1w1[...], ds1b1[...], ds1w2[...], ds1b2[...])
    z0 = mpost + n2_ref[...] * jnp.exp(ls0)
    s_self = s_self + jnp.sum((h0 - z0) ** 2)
    mt0 = (h0 + s0_ref[...]) * inv_c
    s_kl = s_kl + jnp.sum(-1.0 - 2.0 * ls0 + (mpost - mt0) ** 2
                          + jnp.exp(2.0 * ls0))
    t = jnp.maximum(jnp.dot(h1, rd1w1[...], preferred_element_type=jnp.float32)
                    + rd1b1[...], 0.0)
    t = jnp.maximum(jnp.dot(t, rd1w2[...], preferred_element_type=jnp.float32)
                    + rd1b2[...], 0.0)
    rd = jnp.maximum(
        jnp.sum(t * rd1w3[...], axis=1, keepdims=True) + rd1b3[...], 0.0)
    s_deg = s_deg + jnp.sum((rd - deg) ** 2)

    @pl.when(i == 0)
    def _():
        self_ref[...] = jnp.zeros_like(self_ref)
        kl_ref[...] = jnp.zeros_like(kl_ref)
        deg_out_ref[...] = jnp.zeros_like(deg_out_ref)

    self_ref[...] += s_self.reshape(1, 1)
    kl_ref[...] += s_kl.reshape(1, 1)
    deg_out_ref[...] += s_deg.reshape(1, 1)


def _row_spec(d):
    return pl.BlockSpec((BR, d), lambda i: (i, 0))


def _full_spec(shape):
    return pl.BlockSpec(shape, lambda i: (0, 0))


def _decoder(h0, h1, h2, s0, s1, cnt, n1, n2, deg, params):
    p = params
    weights = []
    wspecs = []
    for name, three in (('rs0', False), ('ds0', False), ('rd0', True),
                        ('rs1', False), ('ds1', False), ('dm0', False),
                        ('rd1', True)):
        q = p[name]
        weights += [q['W1'], q['b1'].reshape(1, -1),
                    q['W2'], q['b2'].reshape(1, -1)]
        if three:
            weights += [q['W3'].reshape(1, -1), q['b3'].reshape(1, 1)]
    for w in weights:
        wspecs.append(_full_spec(w.shape))
    out_shape = [jax.ShapeDtypeStruct((1, 1), jnp.float32)] * 3
    out_specs = [pl.BlockSpec((1, 1), lambda i: (0, 0))] * 3
    sums = pl.pallas_call(
        _decoder_body,
        grid=(N // BR,),
        in_specs=[
            _row_spec(F), _row_spec(H1), _row_spec(H2),
            _row_spec(F), _row_spec(H1), _row_spec(1),
            _row_spec(H1), _row_spec(F), _row_spec(1),
        ] + wspecs,
        out_specs=out_specs,
        out_shape=out_shape,
    )(h0, h1, h2, s0, s1, cnt, n1, n2, deg, *weights)
    return sums


def kernel(adj, h0, degree, edge_index, params):
    row = edge_index[0]
    col = edge_index[1]
    n1 = jax.random.normal(jax.random.key(101), (N, H1), jnp.float32)
    n2 = jax.random.normal(jax.random.key(102), (N, F), jnp.float32)

    g0 = params['gin0']
    g1 = params['gin1']
    h1 = _gin_layer(adj, h0, g0['W1'], g0['b1'], g0['W2'], g0['b2'], True)
    s0, s1, cnt = _segment_sums(h0, h1, row, col)
    h2 = _gin_layer(adj, h1, g1['W1'], g1['b1'], g1['W2'], g1['b2'], False)

    s_self, s_kl, s_deg = _decoder(h0, h1, h2, s0, s1, cnt, n1, n2,
                                   degree.reshape(N, 1), params)
    loss_self = s_self[0, 0] / (2.0 * N * 128.0)
    kl = 0.25 * s_kl[0, 0] / (N * 128.0)
    loss_deg = 0.5 * s_deg[0, 0] / N
    loss = loss_self + 1e-4 * kl + 10.0 * loss_deg
    return (loss, h2)
